# baseline (speedup 1.0000x reference)
import functools

import jax
import jax.numpy as jnp
from jax.experimental import pallas as pl
from jax.experimental.pallas import tpu as pltpu


def _fc_kernel(x_ref, b_ref, W_ref, out_ref, m_ref, s_ref, *, NB, rem, V):
    i = pl.program_id(0)
    n = pl.num_programs(0)

    logits = jax.lax.dot_general(
        x_ref[:], W_ref[:],
        dimension_numbers=(((1,), (1,)), ((), ())),
        preferred_element_type=jnp.float32,
    ) + b_ref[:]

    cols = jax.lax.broadcasted_iota(jnp.int32, logits.shape, 1) + i * NB
    masked = jnp.where(cols < V, logits, -jnp.inf)
    m_blk = jnp.max(masked, axis=1, keepdims=True)

    @pl.when(i == 0)
    def _():
        m_ref[:] = jnp.full_like(m_ref, -jnp.inf)
        s_ref[:] = jnp.zeros_like(s_ref)

    m_old = m_ref[:]
    m_new = jnp.maximum(m_old, m_blk)
    s_new = s_ref[:] * jnp.exp(m_old - m_new) + jnp.sum(
        jnp.exp(masked - m_new), axis=1, keepdims=True)
    m_ref[:] = m_new
    s_ref[:] = s_new

    @pl.when(i < n - 1)
    def _():
        out_ref[:, pl.ds(i * NB, NB)] = logits

    @pl.when(i == n - 1)
    def _():
        out_ref[:, pl.ds(i * NB, rem)] = logits[:, :rem]
        lse = m_new + jnp.log(s_new)
        out_ref[:, :] = out_ref[:, :] - lse


@jax.jit
def kernel(x, W, b):
    B, K = x.shape
    V = W.shape[0]
    NB = 2048
    n = pl.cdiv(V, NB)
    rem = V - (n - 1) * NB
    b2 = b.reshape(1, V)

    return pl.pallas_call(
        functools.partial(_fc_kernel, NB=NB, rem=rem, V=V),
        grid=(n,),
        in_specs=[
            pl.BlockSpec((B, K), lambda i: (0, 0)),
            pl.BlockSpec((1, NB), lambda i: (0, i)),
            pl.BlockSpec((NB, K), lambda i: (i, 0)),
        ],
        out_specs=pl.BlockSpec((B, V), lambda i: (0, 0)),
        out_shape=jax.ShapeDtypeStruct((B, V), jnp.float32),
        scratch_shapes=[
            pltpu.VMEM((B, 1), jnp.float32),
            pltpu.VMEM((B, 1), jnp.float32),
        ],
        compiler_params=pltpu.CompilerParams(
            dimension_semantics=("arbitrary",),
        ),
    )(x, b2, W)


# speedup vs baseline: 1.9027x; 1.9027x over previous
import functools

import jax
import jax.numpy as jnp
from jax.experimental import pallas as pl
from jax.experimental.pallas import tpu as pltpu


def _fc_kernel(x_ref, b_ref, W_ref, out_ref, m_ref, s_ref, *, NB, rem, V):
    i = pl.program_id(0)
    n = pl.num_programs(0)

    logits = jax.lax.dot_general(
        x_ref[:], W_ref[:],
        dimension_numbers=(((1,), (1,)), ((), ())),
        preferred_element_type=jnp.float32,
    ) + b_ref[:]

    cols = jax.lax.broadcasted_iota(jnp.int32, logits.shape, 1) + i * NB
    masked = jnp.where(cols < V, logits, -jnp.inf)
    m_blk = jnp.max(masked, axis=1, keepdims=True)

    @pl.when(i == 0)
    def _():
        m_ref[:] = jnp.full_like(m_ref, -jnp.inf)
        s_ref[:] = jnp.zeros_like(s_ref)

    m_old = m_ref[:]
    m_new = jnp.maximum(m_old, m_blk)
    s_new = s_ref[:] * jnp.exp(m_old - m_new) + jnp.sum(
        jnp.exp(masked - m_new), axis=1, keepdims=True)
    m_ref[:] = m_new
    s_ref[:] = s_new

    @pl.when(i < n - 1)
    def _():
        out_ref[:, pl.ds(i * NB, NB)] = logits

    @pl.when(i == n - 1)
    def _():
        out_ref[:, pl.ds(i * NB, rem)] = logits[:, :rem]
        lse = m_new + jnp.log(s_new)
        out_ref[:, :] = out_ref[:, :] - lse


@jax.jit
def kernel(x, W, b):
    B, K = x.shape
    V = W.shape[0]
    NB = 8192
    n = pl.cdiv(V, NB)
    rem = V - (n - 1) * NB
    b2 = b.reshape(1, V)

    return pl.pallas_call(
        functools.partial(_fc_kernel, NB=NB, rem=rem, V=V),
        grid=(n,),
        in_specs=[
            pl.BlockSpec((B, K), lambda i: (0, 0)),
            pl.BlockSpec((1, NB), lambda i: (0, i)),
            pl.BlockSpec((NB, K), lambda i: (i, 0)),
        ],
        out_specs=pl.BlockSpec((B, V), lambda i: (0, 0)),
        out_shape=jax.ShapeDtypeStruct((B, V), jnp.float32),
        scratch_shapes=[
            pltpu.VMEM((B, 1), jnp.float32),
            pltpu.VMEM((B, 1), jnp.float32),
        ],
        compiler_params=pltpu.CompilerParams(
            dimension_semantics=("arbitrary",),
        ),
    )(x, b2, W)


# speedup vs baseline: 2.0513x; 1.0781x over previous
import functools

import jax
import jax.numpy as jnp
from jax.experimental import pallas as pl
from jax.experimental.pallas import tpu as pltpu


def _fc_kernel(x_ref, b_ref, W_ref, out_ref, m_ref, s_ref, *, NB, rem, V):
    i = pl.program_id(0)
    n = pl.num_programs(0)

    logits = jax.lax.dot_general(
        x_ref[:], W_ref[:],
        dimension_numbers=(((1,), (1,)), ((), ())),
        preferred_element_type=jnp.float32,
    ) + b_ref[:]

    cols = jax.lax.broadcasted_iota(jnp.int32, logits.shape, 1) + i * NB
    masked = jnp.where(cols < V, logits, -jnp.inf)
    m_blk = jnp.max(masked, axis=1, keepdims=True)

    @pl.when(i == 0)
    def _():
        m_ref[:] = jnp.full_like(m_ref, -jnp.inf)
        s_ref[:] = jnp.zeros_like(s_ref)

    m_old = m_ref[:]
    m_new = jnp.maximum(m_old, m_blk)
    s_new = s_ref[:] * jnp.exp(m_old - m_new) + jnp.sum(
        jnp.exp(masked - m_new), axis=1, keepdims=True)
    m_ref[:] = m_new
    s_ref[:] = s_new

    @pl.when(i < n - 1)
    def _():
        out_ref[:, pl.ds(i * NB, NB)] = logits

    @pl.when(i == n - 1)
    def _():
        out_ref[:, pl.ds(i * NB, rem)] = logits[:, :rem]
        lse = m_new + jnp.log(s_new)
        out_ref[:, :] = out_ref[:, :] - lse


@jax.jit
def kernel(x, W, b):
    B, K = x.shape
    V = W.shape[0]
    NB = 16384
    n = pl.cdiv(V, NB)
    rem = V - (n - 1) * NB
    b2 = b.reshape(1, V)

    return pl.pallas_call(
        functools.partial(_fc_kernel, NB=NB, rem=rem, V=V),
        grid=(n,),
        in_specs=[
            pl.BlockSpec((B, K), lambda i: (0, 0)),
            pl.BlockSpec((1, NB), lambda i: (0, i)),
            pl.BlockSpec((NB, K), lambda i: (i, 0)),
        ],
        out_specs=pl.BlockSpec((B, V), lambda i: (0, 0)),
        out_shape=jax.ShapeDtypeStruct((B, V), jnp.float32),
        scratch_shapes=[
            pltpu.VMEM((B, 1), jnp.float32),
            pltpu.VMEM((B, 1), jnp.float32),
        ],
        compiler_params=pltpu.CompilerParams(
            dimension_semantics=("arbitrary",),
        ),
    )(x, b2, W)
